# baseline (speedup 1.0000x reference)
import functools

import jax
import jax.numpy as jnp
from jax import lax
from jax.experimental import pallas as pl
from jax.experimental.pallas import tpu as pltpu
from jax.experimental.pallas import tpu_sc as plsc

N = 10000
E = 320000
D = 128

NC = 2
NS = 16
HD = D // NC
ACHUNK = 40
ACHUNKS = 500
AGCHUNKS = 252
PAD_DST = N + 16
NPAD = 10240
NODES_PT = NPAD // NS

_mesh = plsc.VectorSubcoreMesh(core_axis_name="c", subcore_axis_name="s")


@functools.partial(
    pl.kernel,
    out_type=(
        jax.ShapeDtypeStruct((NC, NPAD, HD), jnp.float32),
        jax.ShapeDtypeStruct((NC, NPAD), jnp.float32),
    ),
    mesh=_mesh,
    scratch_types=[
        pltpu.VMEM((ACHUNKS, ACHUNK), jnp.int32),
        pltpu.VMEM((ACHUNKS, ACHUNK), jnp.int32),
        pltpu.VMEM((NODES_PT, HD), jnp.float32),
        pltpu.VMEM((NODES_PT,), jnp.float32),
        pltpu.VMEM((ACHUNK,), jnp.float32),
        pltpu.VMEM((NODES_PT,), jnp.float32),
        pltpu.VMEM_SHARED((NPAD,), jnp.float32),
        pltpu.VMEM_SHARED((NPAD,), jnp.float32),
        pltpu.SemaphoreType.DMA,
        pltpu.SemaphoreType.DMA,
    ],
    compiler_params=pltpu.CompilerParams(
        use_tc_tiling_on_sc=False, needs_layout_passes=False),
)
def _degscale_kernel(xs_hbm, src_hbm, dst_hbm, h_hbm, ideg_hbm,
                     src_v, dst_v, xbuf_v, norm_v, ones_v, zerov_v,
                     odeg_sh, ideg_sh, dsem, xsem):
    c = lax.axis_index("c")
    s = lax.axis_index("s")

    for i in range(ACHUNK // 16):
        ones_v[pl.ds(i * 16, 16)] = jnp.full((16,), 1.0, jnp.float32)

    def zvfill(i, carry):
        zerov_v[pl.ds(i * 16, 16)] = jnp.zeros((16,), jnp.float32)
        return carry
    lax.fori_loop(0, NODES_PT // 16, zvfill, 0)

    pltpu.sync_copy(src_hbm.at[s], src_v)
    pltpu.sync_copy(dst_hbm.at[s], dst_v)
    sl640 = pl.ds(s * NODES_PT, NODES_PT)
    pltpu.async_copy(xs_hbm.at[c, sl640], xbuf_v, xsem)
    pltpu.sync_copy(zerov_v, odeg_sh.at[sl640])
    pltpu.sync_copy(zerov_v, ideg_sh.at[sl640])
    plsc.subcore_barrier()

    def dwait(dst_sh):
        pltpu.make_async_copy(ones_v, dst_sh.at[src_v.at[0]], dsem).wait()

    def obody(j, carry):
        for k in range(10):
            pltpu.async_copy(ones_v, odeg_sh.at[src_v.at[10 * j + k]],
                             dsem, add=True)
        for k in range(10):
            dwait(odeg_sh)
        return carry
    lax.fori_loop(0, ACHUNKS // 10, obody, 0)

    ibase = c * (ACHUNKS // 2)

    def ibody(j, carry):
        for k in range(5):
            pltpu.async_copy(ones_v, ideg_sh.at[dst_v.at[ibase + 5 * j + k]],
                             dsem, add=True)
        for k in range(5):
            dwait(ideg_sh)
        return carry
    lax.fori_loop(0, ACHUNKS // 2 // 5, ibody, 0)
    plsc.subcore_barrier()

    pltpu.sync_copy(ideg_sh.at[sl640], ideg_hbm.at[c, sl640])
    pltpu.sync_copy(odeg_sh.at[sl640], norm_v)

    def nbody(i, carry):
        d = jnp.maximum(norm_v[pl.ds(i * 16, 16)], 1.0)
        u = plsc.bitcast(d, jnp.int32)
        magic = jnp.full((16,), 0x5F3759DF, jnp.int32)
        y = plsc.bitcast(
            magic - lax.shift_right_logical(u, jnp.full((16,), 1, jnp.int32)),
            jnp.float32)
        y = y * (1.5 - 0.5 * d * y * y)
        y = y * (1.5 - 0.5 * d * y * y)
        y = y * (1.5 - 0.5 * d * y * y)
        norm_v[pl.ds(i * 16, 16)] = y
        return carry
    lax.fori_loop(0, NODES_PT // 16, nbody, 0)

    pltpu.make_async_copy(xs_hbm.at[c, sl640], xbuf_v, xsem).wait()

    def sbody(i, carry):
        splat = plsc.load_gather(norm_v, [jnp.full((16,), i, jnp.int32)])
        for k in range(HD // 16):
            ksl = pl.ds(k * 16, 16)
            xbuf_v[i, ksl] = xbuf_v[i, ksl] * splat
        return carry
    lax.fori_loop(0, NODES_PT, sbody, 0)

    pltpu.sync_copy(xbuf_v, h_hbm.at[c, sl640])


@functools.partial(
    pl.kernel,
    out_type=jax.ShapeDtypeStruct((NC, NPAD, HD), jnp.float32),
    mesh=_mesh,
    scratch_types=[
        pltpu.VMEM((ACHUNKS, ACHUNK), jnp.int32),
        pltpu.VMEM((ACHUNKS, ACHUNK), jnp.int32),
        pltpu.VMEM((ACHUNK, HD), jnp.float32),
        pltpu.VMEM((ACHUNK, HD), jnp.float32),
        pltpu.VMEM((64, HD), jnp.float32),
        pltpu.VMEM_SHARED((NPAD, HD), jnp.float32),
        pltpu.SemaphoreType.DMA,
        pltpu.SemaphoreType.DMA,
    ],
    compiler_params=pltpu.CompilerParams(use_tc_tiling_on_sc=False),
)
def _agg_kernel(h_hbm, src_hbm, dst_hbm, out_hbm,
                src_v, dst_v, rows0_v, rows1_v,
                zeros_v, agg_sh, g0, g1):
    c = lax.axis_index("c")
    s = lax.axis_index("s")
    bufs = (rows0_v, rows1_v)
    gsems = (g0, g1)

    def zfill(r, carry):
        for k in range(HD // 16):
            zeros_v[r, pl.ds(k * 16, 16)] = jnp.zeros((16,), jnp.float32)
        return carry
    lax.fori_loop(0, 64, zfill, 0)
    sl640 = pl.ds(s * NODES_PT, NODES_PT)
    for i in range(NODES_PT // 64):
        pltpu.sync_copy(zeros_v, agg_sh.at[pl.ds(s * NODES_PT + i * 64, 64)])
    pltpu.sync_copy(src_hbm.at[s], src_v)
    pltpu.sync_copy(dst_hbm.at[s], dst_v)
    plsc.subcore_barrier()

    hc = h_hbm.at[c]

    def start_g(jj, b):
        pltpu.async_copy(hc.at[src_v.at[jj]], bufs[b], gsems[b])

    def wait_g(b):
        pltpu.make_async_copy(hc.at[src_v.at[0]], bufs[b], gsems[b]).wait()

    PAIRS = ACHUNKS // 2
    start_g(0, 0)

    def body(j, carry):
        start_g(2 * j + 1, 1)
        wait_g(0)
        pltpu.sync_copy(bufs[0], agg_sh.at[dst_v.at[2 * j]], add=True)
        pl.when(j < PAIRS - 1)(functools.partial(start_g, 2 * j + 2, 0))
        wait_g(1)
        pltpu.sync_copy(bufs[1], agg_sh.at[dst_v.at[2 * j + 1]], add=True)
        return carry
    lax.fori_loop(0, PAIRS, body, 0)
    plsc.subcore_barrier()
    pltpu.sync_copy(agg_sh.at[sl640], out_hbm.at[c, sl640])


def _final_body(p_ref, w_ref, b_ref, pi_ref, o_ref):
    deg = pi_ref[0, :N] + pi_ref[1, :N]
    norm = lax.rsqrt(jnp.maximum(deg, 1.0))
    rst = (jnp.dot(p_ref[0, :N, :], w_ref[:HD, :],
                   preferred_element_type=jnp.float32)
           + jnp.dot(p_ref[1, :N, :], w_ref[HD:, :],
                     preferred_element_type=jnp.float32))
    o_ref[...] = rst * norm[:, None] + b_ref[...][None, :]


_final = pl.pallas_call(
    _final_body,
    out_shape=jax.ShapeDtypeStruct((N, D), jnp.float32),
)


def kernel(x, edge_index, W, b):
    src = edge_index[0].astype(jnp.int32)
    dst = edge_index[1].astype(jnp.int32)
    src3 = src.reshape(NS, ACHUNKS, ACHUNK)
    dst3 = dst.reshape(NS, ACHUNKS, ACHUNK)
    xp = jnp.pad(x, ((0, NPAD - N), (0, 0)))
    xs = jnp.stack([xp[:, :HD], xp[:, HD:]])
    h, indeg_p = _degscale_kernel(xs, src3, dst3)
    parts = _agg_kernel(h, src3, dst3)
    return _final(parts, W, b, indeg_p)


# speedup vs baseline: 1.3385x; 1.3385x over previous
import functools

import jax
import jax.numpy as jnp
from jax import lax
from jax.experimental import pallas as pl
from jax.experimental.pallas import tpu as pltpu
from jax.experimental.pallas import tpu_sc as plsc

N = 10000
E = 320000
D = 128

NC = 2
NS = 16
HD = D // NC
ACHUNK = 80
ACHUNKS = 250
NPAD = 10240
NODES_PT = NPAD // NS

_mesh = plsc.VectorSubcoreMesh(core_axis_name="c", subcore_axis_name="s")


@functools.partial(
    pl.kernel,
    out_type=(
        jax.ShapeDtypeStruct((NC, NPAD, HD), jnp.float32),
        jax.ShapeDtypeStruct((NC, NPAD), jnp.float32),
    ),
    mesh=_mesh,
    scratch_types=[
        pltpu.VMEM((ACHUNKS, ACHUNK), jnp.int32),
        pltpu.VMEM((ACHUNKS, ACHUNK), jnp.int32),
        pltpu.VMEM((NODES_PT, HD), jnp.float32),
        pltpu.VMEM((NODES_PT,), jnp.float32),
        pltpu.VMEM((ACHUNK,), jnp.float32),
        pltpu.VMEM((NODES_PT,), jnp.float32),
        pltpu.VMEM_SHARED((NPAD,), jnp.float32),
        pltpu.VMEM_SHARED((NPAD,), jnp.float32),
        pltpu.SemaphoreType.DMA,
        pltpu.SemaphoreType.DMA,
    ],
    compiler_params=pltpu.CompilerParams(
        use_tc_tiling_on_sc=False, needs_layout_passes=False),
)
def _degscale_kernel(xs_hbm, src_hbm, dst_hbm, h_hbm, ideg_hbm,
                     src_v, dst_v, xbuf_v, norm_v, ones_v, zerov_v,
                     odeg_sh, ideg_sh, dsem, xsem):
    c = lax.axis_index("c")
    s = lax.axis_index("s")

    for i in range(ACHUNK // 16):
        ones_v[pl.ds(i * 16, 16)] = jnp.full((16,), 1.0, jnp.float32)

    def zvfill(i, carry):
        zerov_v[pl.ds(i * 16, 16)] = jnp.zeros((16,), jnp.float32)
        return carry
    lax.fori_loop(0, NODES_PT // 16, zvfill, 0)

    pltpu.sync_copy(src_hbm.at[s], src_v)
    pltpu.sync_copy(dst_hbm.at[s], dst_v)
    sl640 = pl.ds(s * NODES_PT, NODES_PT)
    pltpu.async_copy(xs_hbm.at[c, sl640], xbuf_v, xsem)
    pltpu.sync_copy(zerov_v, odeg_sh.at[sl640])
    pltpu.sync_copy(zerov_v, ideg_sh.at[sl640])
    plsc.subcore_barrier()

    def dwait(dst_sh):
        pltpu.make_async_copy(ones_v, dst_sh.at[src_v.at[0]], dsem).wait()

    def obody(j, carry):
        for k in range(10):
            pltpu.async_copy(ones_v, odeg_sh.at[src_v.at[10 * j + k]],
                             dsem, add=True)
        for k in range(10):
            dwait(odeg_sh)
        return carry
    lax.fori_loop(0, ACHUNKS // 10, obody, 0)

    ibase = c * (ACHUNKS // 2)

    def ibody(j, carry):
        for k in range(5):
            pltpu.async_copy(ones_v, ideg_sh.at[dst_v.at[ibase + 5 * j + k]],
                             dsem, add=True)
        for k in range(5):
            dwait(ideg_sh)
        return carry
    lax.fori_loop(0, ACHUNKS // 2 // 5, ibody, 0)
    plsc.subcore_barrier()

    pltpu.sync_copy(ideg_sh.at[sl640], ideg_hbm.at[c, sl640])
    pltpu.sync_copy(odeg_sh.at[sl640], norm_v)

    def nbody(i, carry):
        d = jnp.maximum(norm_v[pl.ds(i * 16, 16)], 1.0)
        u = plsc.bitcast(d, jnp.int32)
        magic = jnp.full((16,), 0x5F3759DF, jnp.int32)
        y = plsc.bitcast(
            magic - lax.shift_right_logical(u, jnp.full((16,), 1, jnp.int32)),
            jnp.float32)
        y = y * (1.5 - 0.5 * d * y * y)
        y = y * (1.5 - 0.5 * d * y * y)
        y = y * (1.5 - 0.5 * d * y * y)
        norm_v[pl.ds(i * 16, 16)] = y
        return carry
    lax.fori_loop(0, NODES_PT // 16, nbody, 0)

    pltpu.make_async_copy(xs_hbm.at[c, sl640], xbuf_v, xsem).wait()

    def sbody(i, carry):
        splat = plsc.load_gather(norm_v, [jnp.full((16,), i, jnp.int32)])
        for k in range(HD // 16):
            ksl = pl.ds(k * 16, 16)
            xbuf_v[i, ksl] = xbuf_v[i, ksl] * splat
        return carry
    lax.fori_loop(0, NODES_PT, sbody, 0)

    pltpu.sync_copy(xbuf_v, h_hbm.at[c, sl640])


@functools.partial(
    pl.kernel,
    out_type=jax.ShapeDtypeStruct((NC, NPAD, HD), jnp.float32),
    mesh=_mesh,
    scratch_types=[
        pltpu.VMEM((ACHUNKS, ACHUNK), jnp.int32),
        pltpu.VMEM((ACHUNKS, ACHUNK), jnp.int32),
        pltpu.VMEM((ACHUNK, HD), jnp.float32),
        pltpu.VMEM((ACHUNK, HD), jnp.float32),
        pltpu.VMEM((64, HD), jnp.float32),
        pltpu.VMEM_SHARED((NPAD, HD), jnp.float32),
        pltpu.SemaphoreType.DMA,
        pltpu.SemaphoreType.DMA,
    ],
    compiler_params=pltpu.CompilerParams(use_tc_tiling_on_sc=False),
)
def _agg_kernel(h_hbm, src_hbm, dst_hbm, out_hbm,
                src_v, dst_v, rows0_v, rows1_v,
                zeros_v, agg_sh, g0, g1):
    c = lax.axis_index("c")
    s = lax.axis_index("s")
    bufs = (rows0_v, rows1_v)
    gsems = (g0, g1)

    def zfill(r, carry):
        for k in range(HD // 16):
            zeros_v[r, pl.ds(k * 16, 16)] = jnp.zeros((16,), jnp.float32)
        return carry
    lax.fori_loop(0, 64, zfill, 0)
    sl640 = pl.ds(s * NODES_PT, NODES_PT)
    for i in range(NODES_PT // 64):
        pltpu.sync_copy(zeros_v, agg_sh.at[pl.ds(s * NODES_PT + i * 64, 64)])
    pltpu.sync_copy(src_hbm.at[s], src_v)
    pltpu.sync_copy(dst_hbm.at[s], dst_v)
    plsc.subcore_barrier()

    hc = h_hbm.at[c]

    def start_g(jj, b):
        pltpu.async_copy(hc.at[src_v.at[jj]], bufs[b], gsems[b])

    def wait_g(b):
        pltpu.make_async_copy(hc.at[src_v.at[0]], bufs[b], gsems[b]).wait()

    PAIRS = ACHUNKS // 2
    start_g(0, 0)

    def body(j, carry):
        start_g(2 * j + 1, 1)
        wait_g(0)
        pltpu.sync_copy(bufs[0], agg_sh.at[dst_v.at[2 * j]], add=True)
        pl.when(j < PAIRS - 1)(functools.partial(start_g, 2 * j + 2, 0))
        wait_g(1)
        pltpu.sync_copy(bufs[1], agg_sh.at[dst_v.at[2 * j + 1]], add=True)
        return carry
    lax.fori_loop(0, PAIRS, body, 0)
    plsc.subcore_barrier()
    pltpu.sync_copy(agg_sh.at[sl640], out_hbm.at[c, sl640])


def _final_body(p_ref, w_ref, b_ref, pi_ref, o_ref):
    deg = pi_ref[0, :N] + pi_ref[1, :N]
    norm = lax.rsqrt(jnp.maximum(deg, 1.0))
    rst = (jnp.dot(p_ref[0, :N, :], w_ref[:HD, :],
                   preferred_element_type=jnp.float32)
           + jnp.dot(p_ref[1, :N, :], w_ref[HD:, :],
                     preferred_element_type=jnp.float32))
    o_ref[...] = rst * norm[:, None] + b_ref[...][None, :]


_final = pl.pallas_call(
    _final_body,
    out_shape=jax.ShapeDtypeStruct((N, D), jnp.float32),
)


def kernel(x, edge_index, W, b):
    src = edge_index[0].astype(jnp.int32)
    dst = edge_index[1].astype(jnp.int32)
    src3 = src.reshape(NS, ACHUNKS, ACHUNK)
    dst3 = dst.reshape(NS, ACHUNKS, ACHUNK)
    xp = jnp.pad(x, ((0, NPAD - N), (0, 0)))
    xs = jnp.stack([xp[:, :HD], xp[:, HD:]])
    h, indeg_p = _degscale_kernel(xs, src3, dst3)
    parts = _agg_kernel(h, src3, dst3)
    return _final(parts, W, b, indeg_p)


# speedup vs baseline: 1.4316x; 1.0695x over previous
import functools

import jax
import jax.numpy as jnp
from jax import lax
from jax.experimental import pallas as pl
from jax.experimental.pallas import tpu as pltpu
from jax.experimental.pallas import tpu_sc as plsc

N = 10000
E = 320000
D = 128

NC = 2
NS = 16
HD = D // NC
ACHUNK = 80
ACHUNKS = 250
NPAD = 10240
NODES_PT = NPAD // NS

_mesh = plsc.VectorSubcoreMesh(core_axis_name="c", subcore_axis_name="s")


@functools.partial(
    pl.kernel,
    out_type=(
        jax.ShapeDtypeStruct((NC, NPAD, HD), jnp.float32),
        jax.ShapeDtypeStruct((NC, NPAD), jnp.float32),
    ),
    mesh=_mesh,
    scratch_types=[
        pltpu.VMEM((ACHUNKS, ACHUNK), jnp.int32),
        pltpu.VMEM((ACHUNKS, ACHUNK), jnp.int32),
        pltpu.VMEM((NODES_PT, HD), jnp.float32),
        pltpu.VMEM((NODES_PT,), jnp.float32),
        pltpu.VMEM((ACHUNK,), jnp.float32),
        pltpu.VMEM((NODES_PT,), jnp.float32),
        pltpu.VMEM_SHARED((NPAD,), jnp.float32),
        pltpu.VMEM_SHARED((NPAD,), jnp.float32),
        pltpu.SemaphoreType.DMA,
        pltpu.SemaphoreType.DMA,
    ],
    compiler_params=pltpu.CompilerParams(
        use_tc_tiling_on_sc=False, needs_layout_passes=False),
)
def _degscale_kernel(xs_hbm, src_hbm, dst_hbm, h_hbm, ideg_hbm,
                     src_v, dst_v, xbuf_v, norm_v, ones_v, zerov_v,
                     odeg_sh, ideg_sh, dsem, xsem):
    c = lax.axis_index("c")
    s = lax.axis_index("s")

    for i in range(ACHUNK // 16):
        ones_v[pl.ds(i * 16, 16)] = jnp.full((16,), 1.0, jnp.float32)

    def zvfill(i, carry):
        zerov_v[pl.ds(i * 16, 16)] = jnp.zeros((16,), jnp.float32)
        return carry
    lax.fori_loop(0, NODES_PT // 16, zvfill, 0)

    pltpu.sync_copy(src_hbm.at[s], src_v)
    pltpu.sync_copy(dst_hbm.at[s], dst_v)
    sl640 = pl.ds(s * NODES_PT, NODES_PT)
    csl = pl.ds(c * HD, HD)
    LAST = (N // NODES_PT) * NODES_PT
    TAIL = N - LAST
    def _xload_full():
        pltpu.async_copy(xs_hbm.at[sl640, csl], xbuf_v, xsem)

    def _xload_tail():
        pltpu.async_copy(xs_hbm.at[pl.ds(LAST, TAIL), csl],
                         xbuf_v.at[pl.ds(0, TAIL)], xsem)

    pl.when(s < NS - 1)(_xload_full)
    pl.when(s == NS - 1)(_xload_tail)
    pltpu.sync_copy(zerov_v, odeg_sh.at[sl640])
    pltpu.sync_copy(zerov_v, ideg_sh.at[sl640])
    plsc.subcore_barrier()

    def dwait(dst_sh):
        pltpu.make_async_copy(ones_v, dst_sh.at[src_v.at[0]], dsem).wait()

    def obody(j, carry):
        for k in range(10):
            pltpu.async_copy(ones_v, odeg_sh.at[src_v.at[10 * j + k]],
                             dsem, add=True)
        for k in range(10):
            dwait(odeg_sh)
        return carry
    lax.fori_loop(0, ACHUNKS // 10, obody, 0)

    ibase = c * (ACHUNKS // 2)

    def ibody(j, carry):
        for k in range(5):
            pltpu.async_copy(ones_v, ideg_sh.at[dst_v.at[ibase + 5 * j + k]],
                             dsem, add=True)
        for k in range(5):
            dwait(ideg_sh)
        return carry
    lax.fori_loop(0, ACHUNKS // 2 // 5, ibody, 0)
    plsc.subcore_barrier()

    pltpu.sync_copy(ideg_sh.at[sl640], ideg_hbm.at[c, sl640])
    pltpu.sync_copy(odeg_sh.at[sl640], norm_v)

    def nbody(i, carry):
        d = jnp.maximum(norm_v[pl.ds(i * 16, 16)], 1.0)
        u = plsc.bitcast(d, jnp.int32)
        magic = jnp.full((16,), 0x5F3759DF, jnp.int32)
        y = plsc.bitcast(
            magic - lax.shift_right_logical(u, jnp.full((16,), 1, jnp.int32)),
            jnp.float32)
        y = y * (1.5 - 0.5 * d * y * y)
        y = y * (1.5 - 0.5 * d * y * y)
        y = y * (1.5 - 0.5 * d * y * y)
        norm_v[pl.ds(i * 16, 16)] = y
        return carry
    lax.fori_loop(0, NODES_PT // 16, nbody, 0)

    def _xwait_full():
        pltpu.make_async_copy(xs_hbm.at[sl640, csl], xbuf_v, xsem).wait()

    def _xwait_tail():
        pltpu.make_async_copy(xs_hbm.at[pl.ds(LAST, TAIL), csl],
                              xbuf_v.at[pl.ds(0, TAIL)], xsem).wait()

    pl.when(s < NS - 1)(_xwait_full)
    pl.when(s == NS - 1)(_xwait_tail)

    def sbody(i, carry):
        splat = plsc.load_gather(norm_v, [jnp.full((16,), i, jnp.int32)])
        for k in range(HD // 16):
            ksl = pl.ds(k * 16, 16)
            xbuf_v[i, ksl] = xbuf_v[i, ksl] * splat
        return carry
    lax.fori_loop(0, NODES_PT, sbody, 0)

    pltpu.sync_copy(xbuf_v, h_hbm.at[c, sl640])


@functools.partial(
    pl.kernel,
    out_type=jax.ShapeDtypeStruct((NC, NPAD, HD), jnp.float32),
    mesh=_mesh,
    scratch_types=[
        pltpu.VMEM((ACHUNKS, ACHUNK), jnp.int32),
        pltpu.VMEM((ACHUNKS, ACHUNK), jnp.int32),
        pltpu.VMEM((ACHUNK, HD), jnp.float32),
        pltpu.VMEM((ACHUNK, HD), jnp.float32),
        pltpu.VMEM((64, HD), jnp.float32),
        pltpu.VMEM_SHARED((NPAD, HD), jnp.float32),
        pltpu.SemaphoreType.DMA,
        pltpu.SemaphoreType.DMA,
    ],
    compiler_params=pltpu.CompilerParams(use_tc_tiling_on_sc=False),
)
def _agg_kernel(h_hbm, src_hbm, dst_hbm, out_hbm,
                src_v, dst_v, rows0_v, rows1_v,
                zeros_v, agg_sh, g0, g1):
    c = lax.axis_index("c")
    s = lax.axis_index("s")
    bufs = (rows0_v, rows1_v)
    gsems = (g0, g1)

    def zfill(r, carry):
        for k in range(HD // 16):
            zeros_v[r, pl.ds(k * 16, 16)] = jnp.zeros((16,), jnp.float32)
        return carry
    lax.fori_loop(0, 64, zfill, 0)
    sl640 = pl.ds(s * NODES_PT, NODES_PT)
    for i in range(NODES_PT // 64):
        pltpu.sync_copy(zeros_v, agg_sh.at[pl.ds(s * NODES_PT + i * 64, 64)])
    pltpu.sync_copy(src_hbm.at[s], src_v)
    pltpu.sync_copy(dst_hbm.at[s], dst_v)
    plsc.subcore_barrier()

    hc = h_hbm.at[c]

    def start_g(jj, b):
        pltpu.async_copy(hc.at[src_v.at[jj]], bufs[b], gsems[b])

    def wait_g(b):
        pltpu.make_async_copy(hc.at[src_v.at[0]], bufs[b], gsems[b]).wait()

    PAIRS = ACHUNKS // 2
    start_g(0, 0)

    def body(j, carry):
        start_g(2 * j + 1, 1)
        wait_g(0)
        pltpu.sync_copy(bufs[0], agg_sh.at[dst_v.at[2 * j]], add=True)
        pl.when(j < PAIRS - 1)(functools.partial(start_g, 2 * j + 2, 0))
        wait_g(1)
        pltpu.sync_copy(bufs[1], agg_sh.at[dst_v.at[2 * j + 1]], add=True)
        return carry
    lax.fori_loop(0, PAIRS, body, 0)
    plsc.subcore_barrier()
    pltpu.sync_copy(agg_sh.at[sl640], out_hbm.at[c, sl640])


def _final_body(p_ref, w_ref, b_ref, pi_ref, o_ref):
    deg = pi_ref[0, :N] + pi_ref[1, :N]
    norm = lax.rsqrt(jnp.maximum(deg, 1.0))
    rst = (jnp.dot(p_ref[0, :N, :], w_ref[:HD, :],
                   preferred_element_type=jnp.float32)
           + jnp.dot(p_ref[1, :N, :], w_ref[HD:, :],
                     preferred_element_type=jnp.float32))
    o_ref[...] = rst * norm[:, None] + b_ref[...][None, :]


_final = pl.pallas_call(
    _final_body,
    out_shape=jax.ShapeDtypeStruct((N, D), jnp.float32),
)


def kernel(x, edge_index, W, b):
    src = edge_index[0].astype(jnp.int32)
    dst = edge_index[1].astype(jnp.int32)
    src3 = src.reshape(NS, ACHUNKS, ACHUNK)
    dst3 = dst.reshape(NS, ACHUNKS, ACHUNK)
    h, indeg_p = _degscale_kernel(x, src3, dst3)
    parts = _agg_kernel(h, src3, dst3)
    return _final(parts, W, b, indeg_p)


# speedup vs baseline: 1.4445x; 1.0090x over previous
import functools

import jax
import jax.numpy as jnp
from jax import lax
from jax.experimental import pallas as pl
from jax.experimental.pallas import tpu as pltpu
from jax.experimental.pallas import tpu_sc as plsc

N = 10000
E = 320000
D = 128

NC = 2
NS = 16
HD = D // NC
ACHUNK = 80
ACHUNKS = 250
NPAD = 10240
NODES_PT = NPAD // NS

_mesh = plsc.VectorSubcoreMesh(core_axis_name="c", subcore_axis_name="s")


@functools.partial(
    pl.kernel,
    out_type=(
        jax.ShapeDtypeStruct((NC, NPAD, HD), jnp.float32),
        jax.ShapeDtypeStruct((NC, NPAD), jnp.float32),
    ),
    mesh=_mesh,
    scratch_types=[
        pltpu.VMEM((ACHUNKS, ACHUNK), jnp.int32),
        pltpu.VMEM((ACHUNKS, ACHUNK), jnp.int32),
        pltpu.VMEM((NODES_PT, HD), jnp.float32),
        pltpu.VMEM((NODES_PT,), jnp.float32),
        pltpu.VMEM((ACHUNK,), jnp.float32),
        pltpu.VMEM((NODES_PT,), jnp.float32),
        pltpu.VMEM_SHARED((NPAD,), jnp.float32),
        pltpu.VMEM_SHARED((NPAD,), jnp.float32),
        pltpu.SemaphoreType.DMA,
        pltpu.SemaphoreType.DMA,
    ],
    compiler_params=pltpu.CompilerParams(
        use_tc_tiling_on_sc=False, needs_layout_passes=False),
)
def _degscale_kernel(xs_hbm, src_hbm, dst_hbm, h_hbm, ideg_hbm,
                     src_v, dst_v, xbuf_v, norm_v, ones_v, zerov_v,
                     odeg_sh, ideg_sh, dsem, xsem):
    c = lax.axis_index("c")
    s = lax.axis_index("s")

    for i in range(ACHUNK // 16):
        ones_v[pl.ds(i * 16, 16)] = jnp.full((16,), 1.0, jnp.float32)

    def zvfill(i, carry):
        zerov_v[pl.ds(i * 16, 16)] = jnp.zeros((16,), jnp.float32)
        return carry
    lax.fori_loop(0, NODES_PT // 16, zvfill, 0)

    pltpu.sync_copy(src_hbm.at[s], src_v)
    pltpu.sync_copy(dst_hbm.at[s], dst_v)
    sl640 = pl.ds(s * NODES_PT, NODES_PT)
    csl = pl.ds(c * HD, HD)
    LAST = (N // NODES_PT) * NODES_PT
    TAIL = N - LAST
    def _xload_full():
        pltpu.async_copy(xs_hbm.at[sl640, csl], xbuf_v, xsem)

    def _xload_tail():
        pltpu.async_copy(xs_hbm.at[pl.ds(LAST, TAIL), csl],
                         xbuf_v.at[pl.ds(0, TAIL)], xsem)

    pl.when(s < NS - 1)(_xload_full)
    pl.when(s == NS - 1)(_xload_tail)
    pltpu.sync_copy(zerov_v, odeg_sh.at[sl640])
    pltpu.sync_copy(zerov_v, ideg_sh.at[sl640])
    plsc.subcore_barrier()

    def dwait(dst_sh):
        pltpu.make_async_copy(ones_v, dst_sh.at[src_v.at[0]], dsem).wait()

    def obody(j, carry):
        for k in range(25):
            pltpu.async_copy(ones_v, odeg_sh.at[src_v.at[25 * j + k]],
                             dsem, add=True)
        for k in range(25):
            dwait(odeg_sh)
        return carry
    lax.fori_loop(0, ACHUNKS // 25, obody, 0)

    ibase = c * (ACHUNKS // 2)

    def ibody(j, carry):
        for k in range(25):
            pltpu.async_copy(ones_v, ideg_sh.at[dst_v.at[ibase + 25 * j + k]],
                             dsem, add=True)
        for k in range(25):
            dwait(ideg_sh)
        return carry
    lax.fori_loop(0, ACHUNKS // 2 // 25, ibody, 0)
    plsc.subcore_barrier()

    pltpu.sync_copy(ideg_sh.at[sl640], ideg_hbm.at[c, sl640])
    pltpu.sync_copy(odeg_sh.at[sl640], norm_v)

    def nbody(i, carry):
        d = jnp.maximum(norm_v[pl.ds(i * 16, 16)], 1.0)
        u = plsc.bitcast(d, jnp.int32)
        magic = jnp.full((16,), 0x5F3759DF, jnp.int32)
        y = plsc.bitcast(
            magic - lax.shift_right_logical(u, jnp.full((16,), 1, jnp.int32)),
            jnp.float32)
        y = y * (1.5 - 0.5 * d * y * y)
        y = y * (1.5 - 0.5 * d * y * y)
        y = y * (1.5 - 0.5 * d * y * y)
        norm_v[pl.ds(i * 16, 16)] = y
        return carry
    lax.fori_loop(0, NODES_PT // 16, nbody, 0)

    def _xwait_full():
        pltpu.make_async_copy(xs_hbm.at[sl640, csl], xbuf_v, xsem).wait()

    def _xwait_tail():
        pltpu.make_async_copy(xs_hbm.at[pl.ds(LAST, TAIL), csl],
                              xbuf_v.at[pl.ds(0, TAIL)], xsem).wait()

    pl.when(s < NS - 1)(_xwait_full)
    pl.when(s == NS - 1)(_xwait_tail)

    def sbody(i, carry):
        splat = plsc.load_gather(norm_v, [jnp.full((16,), i, jnp.int32)])
        for k in range(HD // 16):
            ksl = pl.ds(k * 16, 16)
            xbuf_v[i, ksl] = xbuf_v[i, ksl] * splat
        return carry
    lax.fori_loop(0, NODES_PT, sbody, 0)

    pltpu.sync_copy(xbuf_v, h_hbm.at[c, sl640])


@functools.partial(
    pl.kernel,
    out_type=jax.ShapeDtypeStruct((NC, NPAD, HD), jnp.float32),
    mesh=_mesh,
    scratch_types=[
        pltpu.VMEM((ACHUNKS, ACHUNK), jnp.int32),
        pltpu.VMEM((ACHUNKS, ACHUNK), jnp.int32),
        pltpu.VMEM((ACHUNK, HD), jnp.float32),
        pltpu.VMEM((ACHUNK, HD), jnp.float32),
        pltpu.VMEM((64, HD), jnp.float32),
        pltpu.VMEM_SHARED((NPAD, HD), jnp.float32),
        pltpu.SemaphoreType.DMA,
        pltpu.SemaphoreType.DMA,
    ],
    compiler_params=pltpu.CompilerParams(use_tc_tiling_on_sc=False),
)
def _agg_kernel(h_hbm, src_hbm, dst_hbm, out_hbm,
                src_v, dst_v, rows0_v, rows1_v,
                zeros_v, agg_sh, g0, g1):
    c = lax.axis_index("c")
    s = lax.axis_index("s")
    bufs = (rows0_v, rows1_v)
    gsems = (g0, g1)

    def zfill(r, carry):
        for k in range(HD // 16):
            zeros_v[r, pl.ds(k * 16, 16)] = jnp.zeros((16,), jnp.float32)
        return carry
    lax.fori_loop(0, 64, zfill, 0)
    sl640 = pl.ds(s * NODES_PT, NODES_PT)
    for i in range(NODES_PT // 64):
        pltpu.sync_copy(zeros_v, agg_sh.at[pl.ds(s * NODES_PT + i * 64, 64)])
    pltpu.sync_copy(src_hbm.at[s], src_v)
    pltpu.sync_copy(dst_hbm.at[s], dst_v)
    plsc.subcore_barrier()

    hc = h_hbm.at[c]

    def start_g(jj, b):
        pltpu.async_copy(hc.at[src_v.at[jj]], bufs[b], gsems[b])

    def wait_g(b):
        pltpu.make_async_copy(hc.at[src_v.at[0]], bufs[b], gsems[b]).wait()

    PAIRS = ACHUNKS // 2
    start_g(0, 0)

    def body(j, carry):
        start_g(2 * j + 1, 1)
        wait_g(0)
        pltpu.sync_copy(bufs[0], agg_sh.at[dst_v.at[2 * j]], add=True)
        pl.when(j < PAIRS - 1)(functools.partial(start_g, 2 * j + 2, 0))
        wait_g(1)
        pltpu.sync_copy(bufs[1], agg_sh.at[dst_v.at[2 * j + 1]], add=True)
        return carry
    lax.fori_loop(0, PAIRS, body, 0)
    plsc.subcore_barrier()
    pltpu.sync_copy(agg_sh.at[sl640], out_hbm.at[c, sl640])


def _final_body(p_ref, w_ref, b_ref, pi_ref, o_ref):
    deg = pi_ref[0, :N] + pi_ref[1, :N]
    norm = lax.rsqrt(jnp.maximum(deg, 1.0))
    rst = (jnp.dot(p_ref[0, :N, :], w_ref[:HD, :],
                   preferred_element_type=jnp.float32)
           + jnp.dot(p_ref[1, :N, :], w_ref[HD:, :],
                     preferred_element_type=jnp.float32))
    o_ref[...] = rst * norm[:, None] + b_ref[...][None, :]


_final = pl.pallas_call(
    _final_body,
    out_shape=jax.ShapeDtypeStruct((N, D), jnp.float32),
)


def kernel(x, edge_index, W, b):
    src = edge_index[0].astype(jnp.int32)
    dst = edge_index[1].astype(jnp.int32)
    src3 = src.reshape(NS, ACHUNKS, ACHUNK)
    dst3 = dst.reshape(NS, ACHUNKS, ACHUNK)
    h, indeg_p = _degscale_kernel(x, src3, dst3)
    parts = _agg_kernel(h, src3, dst3)
    return _final(parts, W, b, indeg_p)
